# baseline (speedup 1.0000x reference)
import jax
import jax.numpy as jnp
from jax import lax
from jax.experimental import pallas as pl
from jax.experimental.pallas import tpu as pltpu
from jax.experimental.pallas import tpu_sc as plsc

N_NODES = 10000
E_EDGES = 320000
D_FEAT = 128
D_EDGE = 16
D_OUT = 128

NUM_CORES = 2
NUM_SUBCORES = 16
NUM_TILES = NUM_CORES * NUM_SUBCORES
GROUP = 128
LANES = 16
NUM_ROWS = E_EDGES // GROUP
ROWS_TILE = NUM_ROWS // NUM_TILES
EXTRA_BASE = ROWS_TILE * NUM_TILES
NUM_EXTRA = NUM_ROWS - EXTRA_BASE
CHUNK_ROWS = 16
FULL_CHUNKS = ROWS_TILE // CHUNK_ROWS
TAIL_ROWS = ROWS_TILE - FULL_CHUNKS * CHUNK_ROWS
TBL_ROWS = 10240
STRIPE = TBL_ROWS // NUM_SUBCORES


CHUNK_EDGES = CHUNK_ROWS * GROUP
TAIL_EDGES = TAIL_ROWS * GROUP
EDGES_TILE = ROWS_TILE * GROUP


def _sc_scatter(idx_hbm, attr_hbm,
                sums_out, counts_out,
                idx_v, data_v, hist_v, buf_v, cnt_v, cnt16_v,
                sums_sh, hist_sh, sem):
    c = lax.axis_index("c")
    s = lax.axis_index("s")
    t = c * NUM_SUBCORES + s

    z16 = jnp.zeros((LANES,), jnp.float32)

    def zero_data(j, carry):
        data_v[j, pl.ds(0, LANES)] = z16
        return carry

    lax.fori_loop(0, STRIPE, zero_data, 0)

    def zero_hist(i, carry):
        hist_v[pl.ds(i * LANES, LANES)] = z16
        return carry

    lax.fori_loop(0, TBL_ROWS // LANES, zero_hist, 0)
    pltpu.sync_copy(data_v.at[pl.ds(0, STRIPE)],
                    sums_sh.at[pl.ds(s * STRIPE, STRIPE)])
    plsc.subcore_barrier()

    edge0 = t * EDGES_TILE

    def fire_scatters(n_rows):
        def fire(j, carry2):
            pltpu.async_copy(data_v.at[pl.ds(j * GROUP, GROUP)],
                             sums_sh.at[idx_v.at[j]],
                             sem, add=True)
            return carry2
        lax.fori_loop(0, n_rows, fire, 0)

    def hist_rows(n_rows):
        def hrow(j, carry2):
            for k in range(GROUP // LANES):
                idx16 = idx_v[j, pl.ds(k * LANES, LANES)]
                cnt, m = plsc.scan_count(idx16)
                plsc.addupdate_scatter(hist_v, [idx16],
                                       cnt.astype(jnp.float32), mask=m)
            return carry2
        lax.fori_loop(0, n_rows, hrow, 0)

    def drain(n_rows):
        pltpu.make_async_copy(attr_hbm.at[pl.ds(0, n_rows * GROUP)],
                              data_v.at[pl.ds(0, n_rows * GROUP)], sem).wait()

    row0 = t * ROWS_TILE

    def chunk_body(ch, carry):
        r = row0 + ch * CHUNK_ROWS
        e = edge0 + ch * CHUNK_EDGES
        pltpu.sync_copy(idx_hbm.at[pl.ds(r, CHUNK_ROWS)], idx_v)
        pltpu.sync_copy(attr_hbm.at[pl.ds(e, CHUNK_EDGES)], data_v)
        fire_scatters(CHUNK_ROWS)
        hist_rows(CHUNK_ROWS)
        drain(CHUNK_ROWS)
        return carry

    lax.fori_loop(0, FULL_CHUNKS, chunk_body, 0)

    r_tail = row0 + FULL_CHUNKS * CHUNK_ROWS
    e_tail = edge0 + FULL_CHUNKS * CHUNK_EDGES
    pltpu.sync_copy(idx_hbm.at[pl.ds(r_tail, TAIL_ROWS)],
                    idx_v.at[pl.ds(0, TAIL_ROWS)])
    pltpu.sync_copy(attr_hbm.at[pl.ds(e_tail, TAIL_EDGES)],
                    data_v.at[pl.ds(0, TAIL_EDGES)])
    fire_scatters(TAIL_ROWS)
    hist_rows(TAIL_ROWS)
    drain(TAIL_ROWS)

    @pl.when(t < NUM_EXTRA)
    def _():
        r_x = EXTRA_BASE + t
        pltpu.sync_copy(idx_hbm.at[pl.ds(r_x, 1)], idx_v.at[pl.ds(0, 1)])
        pltpu.sync_copy(attr_hbm.at[pl.ds(r_x * GROUP, GROUP)],
                        data_v.at[pl.ds(0, GROUP)])
        fire_scatters(1)
        hist_rows(1)
        drain(1)

    plsc.subcore_barrier()

    pltpu.sync_copy(hist_v, hist_sh.at[s])
    plsc.subcore_barrier()
    for r in range(NUM_SUBCORES):
        pltpu.sync_copy(hist_sh.at[r].at[pl.ds(s * STRIPE, STRIPE)],
                        buf_v.at[r])

    def reduce_g(g, carry):
        acc = buf_v[0, pl.ds(g * LANES, LANES)]
        for r in range(1, NUM_SUBCORES):
            acc = acc + buf_v[r, pl.ds(g * LANES, LANES)]
        cnt_v[pl.ds(g * LANES, LANES)] = acc
        return carry

    lax.fori_loop(0, STRIPE // LANES, reduce_g, 0)

    def repl_n(n, carry):
        v16 = plsc.load_gather(cnt_v, [jnp.broadcast_to(n, (LANES,))])
        cnt16_v[pl.ds(n * LANES, LANES)] = v16
        return carry

    lax.fori_loop(0, STRIPE, repl_n, 0)
    pltpu.sync_copy(
        cnt16_v,
        counts_out.at[c].at[pl.ds(s * STRIPE * LANES, STRIPE * LANES)])
    pltpu.sync_copy(sums_sh.at[pl.ds(s * STRIPE, STRIPE)],
                    sums_out.at[c].at[pl.ds(s * STRIPE, STRIPE)])


PACK_ROWS = TBL_ROWS * D_EDGE // 128


def _tc_edge(sums_ref, counts_ref, web_ref, out_ref):
    sp = sums_ref[0] + sums_ref[1]
    cp = counts_ref[0] + counts_ref[1]
    recv = sp / jnp.maximum(cp, 1.0)
    out_ref[...] = jnp.dot(recv, web_ref[...],
                           preferred_element_type=jnp.float32)


def _tc_final(x_ref, e_ref, wx_ref, b_ref, out_ref):
    acc = jnp.dot(x_ref[...], wx_ref[...], preferred_element_type=jnp.float32)
    out_ref[...] = acc + e_ref[...] + b_ref[...]


BLK = 2000


def kernel(x, edge_index, edge_attr, W, b):
    dst = edge_index[1].astype(jnp.int32)
    attr = edge_attr.astype(jnp.float32)

    mesh = plsc.VectorSubcoreMesh(core_axis_name="c", subcore_axis_name="s")
    scatter_call = pl.kernel(
        _sc_scatter,
        mesh=mesh,
        out_type=[
            jax.ShapeDtypeStruct((NUM_CORES, TBL_ROWS, D_EDGE), jnp.float32),
            jax.ShapeDtypeStruct((NUM_CORES, TBL_ROWS * D_EDGE), jnp.float32),
        ],
        scratch_types=[
            pltpu.VMEM((CHUNK_ROWS, GROUP), jnp.int32),
            pltpu.VMEM((CHUNK_EDGES, D_EDGE), jnp.float32),
            pltpu.VMEM((TBL_ROWS,), jnp.float32),
            pltpu.VMEM((NUM_SUBCORES, STRIPE), jnp.float32),
            pltpu.VMEM((STRIPE,), jnp.float32),
            pltpu.VMEM((STRIPE * LANES,), jnp.float32),
            pltpu.VMEM_SHARED((TBL_ROWS, D_EDGE), jnp.float32),
            pltpu.VMEM_SHARED((NUM_SUBCORES, TBL_ROWS), jnp.float32),
            pltpu.SemaphoreType.DMA,
        ],
        compiler_params=pltpu.CompilerParams(use_tc_tiling_on_sc=False,
                                             needs_layout_passes=False),
    )
    sums, counts = scatter_call(dst.reshape(NUM_ROWS, GROUP), attr)

    wx = W[:D_FEAT]
    we = W[D_FEAT:]
    b2 = b.reshape(1, D_OUT)
    web = jnp.kron(jnp.eye(8, dtype=jnp.float32), we)
    sums_p = sums.reshape(NUM_CORES, PACK_ROWS, 128)
    counts_p = counts.reshape(NUM_CORES, PACK_ROWS, 128)
    eout_p = pl.pallas_call(
        _tc_edge,
        out_shape=jax.ShapeDtypeStruct((PACK_ROWS, 8 * D_OUT), jnp.float32),
    )(sums_p, counts_p, web)
    eout = eout_p.reshape(TBL_ROWS, D_OUT)
    out = pl.pallas_call(
        _tc_final,
        grid=(N_NODES // BLK,),
        in_specs=[
            pl.BlockSpec((BLK, D_FEAT), lambda i: (i, 0)),
            pl.BlockSpec((BLK, D_OUT), lambda i: (i, 0)),
            pl.BlockSpec((D_FEAT, D_OUT), lambda i: (0, 0)),
            pl.BlockSpec((1, D_OUT), lambda i: (0, 0)),
        ],
        out_specs=pl.BlockSpec((BLK, D_OUT), lambda i: (i, 0)),
        out_shape=jax.ShapeDtypeStruct((N_NODES, D_OUT), jnp.float32),
    )(x, eout, wx, b2)
    return out


# speedup vs baseline: 1.0146x; 1.0146x over previous
import jax
import jax.numpy as jnp
from jax import lax
from jax.experimental import pallas as pl
from jax.experimental.pallas import tpu as pltpu
from jax.experimental.pallas import tpu_sc as plsc

N_NODES = 10000
E_EDGES = 320000
D_FEAT = 128
D_EDGE = 16
D_OUT = 128

NUM_CORES = 2
NUM_SUBCORES = 16
NUM_TILES = NUM_CORES * NUM_SUBCORES
GROUP = 128
LANES = 16
NUM_ROWS = E_EDGES // GROUP
ROWS_TILE = NUM_ROWS // NUM_TILES
EXTRA_BASE = ROWS_TILE * NUM_TILES
NUM_EXTRA = NUM_ROWS - EXTRA_BASE
CHUNK_ROWS = 32
FULL_CHUNKS = ROWS_TILE // CHUNK_ROWS
TAIL_ROWS = ROWS_TILE - FULL_CHUNKS * CHUNK_ROWS
TBL_ROWS = 10240
STRIPE = TBL_ROWS // NUM_SUBCORES


CHUNK_EDGES = CHUNK_ROWS * GROUP
TAIL_EDGES = TAIL_ROWS * GROUP
EDGES_TILE = ROWS_TILE * GROUP


def _sc_scatter(idx_hbm, attr_hbm,
                sums_out, counts_out,
                idx_v, data_v, hist_v, buf_v, cnt_v, cnt16_v,
                sums_sh, hist_sh, sem):
    c = lax.axis_index("c")
    s = lax.axis_index("s")
    t = c * NUM_SUBCORES + s

    z16 = jnp.zeros((LANES,), jnp.float32)

    def zero_data(j, carry):
        data_v[j, pl.ds(0, LANES)] = z16
        return carry

    lax.fori_loop(0, STRIPE, zero_data, 0)

    def zero_hist(i, carry):
        hist_v[pl.ds(i * LANES, LANES)] = z16
        return carry

    lax.fori_loop(0, TBL_ROWS // LANES, zero_hist, 0)
    pltpu.sync_copy(data_v.at[pl.ds(0, STRIPE)],
                    sums_sh.at[pl.ds(s * STRIPE, STRIPE)])
    plsc.subcore_barrier()

    edge0 = t * EDGES_TILE

    def fire_scatters(n_rows):
        def fire(j, carry2):
            pltpu.async_copy(data_v.at[pl.ds(j * GROUP, GROUP)],
                             sums_sh.at[idx_v.at[j]],
                             sem, add=True)
            return carry2
        lax.fori_loop(0, n_rows, fire, 0)

    def hist_rows(n_rows):
        def hrow(j, carry2):
            for k in range(GROUP // LANES):
                idx16 = idx_v[j, pl.ds(k * LANES, LANES)]
                cnt, m = plsc.scan_count(idx16)
                plsc.addupdate_scatter(hist_v, [idx16],
                                       cnt.astype(jnp.float32), mask=m)
            return carry2
        lax.fori_loop(0, n_rows, hrow, 0)

    def drain(n_rows):
        pltpu.make_async_copy(attr_hbm.at[pl.ds(0, n_rows * GROUP)],
                              data_v.at[pl.ds(0, n_rows * GROUP)], sem).wait()

    row0 = t * ROWS_TILE

    def chunk_body(ch, carry):
        r = row0 + ch * CHUNK_ROWS
        e = edge0 + ch * CHUNK_EDGES
        pltpu.sync_copy(idx_hbm.at[pl.ds(r, CHUNK_ROWS)], idx_v)
        pltpu.sync_copy(attr_hbm.at[pl.ds(e, CHUNK_EDGES)], data_v)
        fire_scatters(CHUNK_ROWS)
        hist_rows(CHUNK_ROWS)
        drain(CHUNK_ROWS)
        return carry

    lax.fori_loop(0, FULL_CHUNKS, chunk_body, 0)

    r_tail = row0 + FULL_CHUNKS * CHUNK_ROWS
    e_tail = edge0 + FULL_CHUNKS * CHUNK_EDGES
    pltpu.sync_copy(idx_hbm.at[pl.ds(r_tail, TAIL_ROWS)],
                    idx_v.at[pl.ds(0, TAIL_ROWS)])
    pltpu.sync_copy(attr_hbm.at[pl.ds(e_tail, TAIL_EDGES)],
                    data_v.at[pl.ds(0, TAIL_EDGES)])
    fire_scatters(TAIL_ROWS)
    hist_rows(TAIL_ROWS)
    drain(TAIL_ROWS)

    @pl.when(t < NUM_EXTRA)
    def _():
        r_x = EXTRA_BASE + t
        pltpu.sync_copy(idx_hbm.at[pl.ds(r_x, 1)], idx_v.at[pl.ds(0, 1)])
        pltpu.sync_copy(attr_hbm.at[pl.ds(r_x * GROUP, GROUP)],
                        data_v.at[pl.ds(0, GROUP)])
        fire_scatters(1)
        hist_rows(1)
        drain(1)

    plsc.subcore_barrier()

    pltpu.sync_copy(hist_v, hist_sh.at[s])
    plsc.subcore_barrier()
    for r in range(NUM_SUBCORES):
        pltpu.sync_copy(hist_sh.at[r].at[pl.ds(s * STRIPE, STRIPE)],
                        buf_v.at[r])

    def reduce_g(g, carry):
        acc = buf_v[0, pl.ds(g * LANES, LANES)]
        for r in range(1, NUM_SUBCORES):
            acc = acc + buf_v[r, pl.ds(g * LANES, LANES)]
        cnt_v[pl.ds(g * LANES, LANES)] = acc
        return carry

    lax.fori_loop(0, STRIPE // LANES, reduce_g, 0)

    def repl_n(n, carry):
        v16 = plsc.load_gather(cnt_v, [jnp.broadcast_to(n, (LANES,))])
        cnt16_v[pl.ds(n * LANES, LANES)] = v16
        return carry

    lax.fori_loop(0, STRIPE, repl_n, 0)
    pltpu.sync_copy(
        cnt16_v,
        counts_out.at[c].at[pl.ds(s * STRIPE * LANES, STRIPE * LANES)])
    pltpu.sync_copy(sums_sh.at[pl.ds(s * STRIPE, STRIPE)],
                    sums_out.at[c].at[pl.ds(s * STRIPE, STRIPE)])


PACK_ROWS = TBL_ROWS * D_EDGE // 128


def _tc_edge(sums_ref, counts_ref, web_ref, out_ref):
    sp = sums_ref[0] + sums_ref[1]
    cp = counts_ref[0] + counts_ref[1]
    recv = sp / jnp.maximum(cp, 1.0)
    out_ref[...] = jnp.dot(recv, web_ref[...],
                           preferred_element_type=jnp.float32)


def _tc_final(x_ref, e_ref, wx_ref, b_ref, out_ref):
    acc = jnp.dot(x_ref[...], wx_ref[...], preferred_element_type=jnp.float32)
    out_ref[...] = acc + e_ref[...] + b_ref[...]


BLK = 2000


def kernel(x, edge_index, edge_attr, W, b):
    dst = edge_index[1].astype(jnp.int32)
    attr = edge_attr.astype(jnp.float32)

    mesh = plsc.VectorSubcoreMesh(core_axis_name="c", subcore_axis_name="s")
    scatter_call = pl.kernel(
        _sc_scatter,
        mesh=mesh,
        out_type=[
            jax.ShapeDtypeStruct((NUM_CORES, TBL_ROWS, D_EDGE), jnp.float32),
            jax.ShapeDtypeStruct((NUM_CORES, TBL_ROWS * D_EDGE), jnp.float32),
        ],
        scratch_types=[
            pltpu.VMEM((CHUNK_ROWS, GROUP), jnp.int32),
            pltpu.VMEM((CHUNK_EDGES, D_EDGE), jnp.float32),
            pltpu.VMEM((TBL_ROWS,), jnp.float32),
            pltpu.VMEM((NUM_SUBCORES, STRIPE), jnp.float32),
            pltpu.VMEM((STRIPE,), jnp.float32),
            pltpu.VMEM((STRIPE * LANES,), jnp.float32),
            pltpu.VMEM_SHARED((TBL_ROWS, D_EDGE), jnp.float32),
            pltpu.VMEM_SHARED((NUM_SUBCORES, TBL_ROWS), jnp.float32),
            pltpu.SemaphoreType.DMA,
        ],
        compiler_params=pltpu.CompilerParams(use_tc_tiling_on_sc=False,
                                             needs_layout_passes=False),
    )
    sums, counts = scatter_call(dst.reshape(NUM_ROWS, GROUP), attr)

    wx = W[:D_FEAT]
    we = W[D_FEAT:]
    b2 = b.reshape(1, D_OUT)
    web = jnp.kron(jnp.eye(8, dtype=jnp.float32), we)
    sums_p = sums.reshape(NUM_CORES, PACK_ROWS, 128)
    counts_p = counts.reshape(NUM_CORES, PACK_ROWS, 128)
    eout_p = pl.pallas_call(
        _tc_edge,
        out_shape=jax.ShapeDtypeStruct((PACK_ROWS, 8 * D_OUT), jnp.float32),
    )(sums_p, counts_p, web)
    eout = eout_p.reshape(TBL_ROWS, D_OUT)
    out = pl.pallas_call(
        _tc_final,
        grid=(N_NODES // BLK,),
        in_specs=[
            pl.BlockSpec((BLK, D_FEAT), lambda i: (i, 0)),
            pl.BlockSpec((BLK, D_OUT), lambda i: (i, 0)),
            pl.BlockSpec((D_FEAT, D_OUT), lambda i: (0, 0)),
            pl.BlockSpec((1, D_OUT), lambda i: (0, 0)),
        ],
        out_specs=pl.BlockSpec((BLK, D_OUT), lambda i: (i, 0)),
        out_shape=jax.ShapeDtypeStruct((N_NODES, D_OUT), jnp.float32),
    )(x, eout, wx, b2)
    return out
